# baseline (speedup 1.0000x reference)
import functools

import jax
import jax.numpy as jnp
from jax import lax
from jax.experimental import pallas as pl
from jax.experimental.pallas import tpu as pltpu
from jax.experimental.pallas import tpu_sc as plsc

_F = 8
_NL = 40
_STREAMS = (128, 128, 64)


@functools.lru_cache(maxsize=None)
def _build(b_total: int, l_total: int, vocab: int, d: int):
    info = plsc.get_sparse_core_info()
    num_workers = info.num_cores * info.num_subcores
    chunk_rows = _NL * _F
    n_rows = b_total * l_total * _F
    assert b_total % num_workers == 0 and l_total % _NL == 0
    b_per_worker = b_total // num_workers
    chunks_per_b = l_total // _NL
    n_chunks = b_per_worker * chunks_per_b
    assert n_chunks % 2 == 0
    assert sum(_STREAMS) == chunk_rows

    mesh = plsc.VectorSubcoreMesh(core_axis_name="c", subcore_axis_name="s")

    @functools.partial(
        pl.kernel,
        mesh=mesh,
        compiler_params=pltpu.CompilerParams(
            use_tc_tiling_on_sc=False, needs_layout_passes=False
        ),
        out_type=jax.ShapeDtypeStruct((n_rows, d), jnp.float32),
        scratch_types=[
            pltpu.VMEM((2, _NL, _F), jnp.int32),
            pltpu.VMEM((2, chunk_rows), jnp.int32),
            pltpu.VMEM((2, chunk_rows, d), jnp.float32),
            pltpu.SemaphoreType.DMA,
            pltpu.SemaphoreType.DMA,
            pltpu.SemaphoreType.DMA,
            pltpu.SemaphoreType.DMA,
        ],
    )
    def gather_kernel(x_hbm, tab_hbm, out_hbm, xs_v, idx_v, rows_v, g0, g1, o0, o1):
        sem_g = [g0, g1]
        sem_o = [o0, o1]
        wid = lax.axis_index("s") * info.num_cores + lax.axis_index("c")
        b_base = wid * b_per_worker
        tab_flat = tab_hbm.at[0]
        lanes = jnp.arange(16, dtype=jnp.int32)
        field_of_lane = lanes & (_F - 1)
        off_vec = field_of_lane * vocab
        row_of_lane = lanes >> 3

        def coords(c):
            return b_base + c // chunks_per_b, (c % chunks_per_b) * _NL

        def prep(c, b):
            bb, l0 = coords(c)
            pltpu.sync_copy(x_hbm.at[bb, pl.ds(l0, _NL)], xs_v.at[b])
            for k in range(chunk_rows // 16):
                vals = plsc.load_gather(
                    xs_v.at[b], [2 * k + row_of_lane, field_of_lane]
                )
                idx_v[b, pl.ds(k * 16, 16)] = vals + off_vec
            pos = 0
            for sz in _STREAMS:
                pltpu.async_copy(
                    tab_flat.at[idx_v.at[b, pl.ds(pos, sz)]],
                    rows_v.at[b, pl.ds(pos, sz)],
                    sem_g[b],
                )
                pos += sz

        def wait_gathers(b):
            pltpu.make_async_copy(
                tab_flat.at[pl.ds(0, chunk_rows)], rows_v.at[b], sem_g[b]
            ).wait()

        def fire_out(c, b):
            bb, l0 = coords(c)
            row0 = pl.multiple_of((bb * l_total + l0) * _F, chunk_rows)
            pltpu.async_copy(rows_v.at[b], out_hbm.at[pl.ds(row0, chunk_rows)], sem_o[b])

        def wait_out(b):
            pltpu.make_async_copy(
                rows_v.at[b], out_hbm.at[pl.ds(0, chunk_rows)], sem_o[b]
            ).wait()

        def pair_body(i, carry):
            c0 = 2 * i

            @pl.when(i > 0)
            def _():
                wait_out(1)
            prep(c0 + 1, 1)
            wait_gathers(0)
            fire_out(c0, 0)

            @pl.when(i < n_chunks // 2 - 1)
            def _():
                wait_out(0)
                prep(c0 + 2, 0)
            wait_gathers(1)
            fire_out(c0 + 1, 1)
            return carry

        prep(0, 0)
        lax.fori_loop(0, n_chunks // 2, pair_body, 0)
        wait_out(0)
        wait_out(1)

    return gather_kernel


def kernel(x, tables):
    b, l, f = x.shape
    n_tab, vocab, d = tables.shape
    out = _build(b, l, vocab, d)(x, tables)
    return out.reshape(b, l, f * d)
